# baseline (speedup 1.0000x reference)
import functools

import jax
import jax.numpy as jnp
from jax import lax
from jax.experimental import pallas as pl
from jax.experimental.pallas import tpu as pltpu
from jax.experimental.pallas import tpu_sc as plsc

D_MODEL = 64
SCALE = 8.0
NUM_CORES = 2
NUM_SUBCORES = 16
NUM_WORKERS = NUM_CORES * NUM_SUBCORES
CHUNK = 512


@functools.partial(jax.jit, static_argnames=("total",))
def _emb(flat_idx, table, total):
    b_per_w = total // NUM_WORKERS
    n_chunks = b_per_w // CHUNK
    mesh = plsc.VectorSubcoreMesh(core_axis_name="c", subcore_axis_name="s")

    @functools.partial(
        pl.kernel,
        mesh=mesh,
        out_type=jax.ShapeDtypeStruct((total, D_MODEL), jnp.float32),
        compiler_params=pltpu.CompilerParams(use_tc_tiling_on_sc=False),
        scratch_types=[
            pltpu.VMEM((CHUNK,), jnp.int32),
            pltpu.VMEM((CHUNK, D_MODEL), jnp.float32),
            pltpu.SemaphoreType.DMA,
        ],
    )
    def emb_kernel(x_hbm, table_hbm, out_hbm, idx_v, rows_v, sem):
        wid = lax.axis_index("s") * NUM_CORES + lax.axis_index("c")
        base = wid * b_per_w

        def chunk_body(g, carry):
            off = base + g * CHUNK
            pltpu.sync_copy(x_hbm.at[pl.ds(off, CHUNK)], idx_v)
            pltpu.async_copy(table_hbm.at[idx_v], rows_v, sem).wait()

            def scale_body(i, c):
                for j in range(D_MODEL // 16):
                    sl = pl.ds(j * 16, 16)
                    rows_v[i, sl] = rows_v[i, sl] * SCALE
                return c

            lax.fori_loop(0, CHUNK, scale_body, 0)
            pltpu.sync_copy(rows_v, out_hbm.at[pl.ds(off, CHUNK)])
            return carry

        lax.fori_loop(0, n_chunks, chunk_body, 0)

    return emb_kernel(flat_idx, table)


def kernel(x, table):
    total = x.shape[0] * x.shape[1]
    flat = x.reshape(total).astype(jnp.int32)
    out = _emb(flat, table, total)
    return out.reshape(x.shape[0], x.shape[1], D_MODEL)


# speedup vs baseline: 1.1375x; 1.1375x over previous
import functools

import jax
import jax.numpy as jnp
from jax import lax
from jax.experimental import pallas as pl
from jax.experimental.pallas import tpu as pltpu
from jax.experimental.pallas import tpu_sc as plsc

D_MODEL = 64
SCALE = 8.0
NUM_CORES = 2
NUM_SUBCORES = 16
NUM_WORKERS = NUM_CORES * NUM_SUBCORES
CHUNK = 512
ROWS_PER_STEP = 4


@functools.partial(jax.jit, static_argnames=("total",))
def _emb(flat_idx, table, total):
    b_per_w = total // NUM_WORKERS
    n_chunks = b_per_w // CHUNK
    assert n_chunks % 2 == 0
    mesh = plsc.VectorSubcoreMesh(core_axis_name="c", subcore_axis_name="s")

    @functools.partial(
        pl.kernel,
        mesh=mesh,
        out_type=jax.ShapeDtypeStruct((total, D_MODEL), jnp.float32),
        compiler_params=pltpu.CompilerParams(use_tc_tiling_on_sc=False),
        scratch_types=[
            pltpu.VMEM((b_per_w,), jnp.int32),
            pltpu.VMEM((CHUNK, D_MODEL), jnp.float32),
            pltpu.VMEM((CHUNK, D_MODEL), jnp.float32),
            pltpu.SemaphoreType.DMA,
            pltpu.SemaphoreType.DMA,
            pltpu.SemaphoreType.DMA,
            pltpu.SemaphoreType.DMA,
        ],
    )
    def emb_kernel(x_hbm, table_hbm, out_hbm, idx_v, rows0, rows1,
                   g0, g1, s0, s1):
        wid = lax.axis_index("s") * NUM_CORES + lax.axis_index("c")
        base = wid * b_per_w
        rows = (rows0, rows1)
        gsem = (g0, g1)
        ssem = (s0, s1)

        def start_gather(c, slot):
            src = table_hbm.at[idx_v.at[pl.ds(c * CHUNK, CHUNK)]]
            pltpu.async_copy(src, rows[slot], gsem[slot])

        def start_scatter(c, slot):
            dst = out_hbm.at[pl.ds(base + c * CHUNK, CHUNK)]
            pltpu.async_copy(rows[slot], dst, ssem[slot])

        def wait_gather(slot):
            pltpu.make_async_copy(
                table_hbm.at[pl.ds(0, CHUNK)], rows[slot], gsem[slot]
            ).wait()

        def wait_scatter(slot):
            pltpu.make_async_copy(
                rows[slot], out_hbm.at[pl.ds(base, CHUNK)], ssem[slot]
            ).wait()

        def scale(slot):
            buf = rows[slot]

            def body(i, carry):
                r = i * ROWS_PER_STEP
                for dr in range(ROWS_PER_STEP):
                    for j in range(D_MODEL // 16):
                        sl = pl.ds(j * 16, 16)
                        buf[r + dr, sl] = buf[r + dr, sl] * SCALE
                return carry

            lax.fori_loop(0, CHUNK // ROWS_PER_STEP, body, 0)

        pltpu.sync_copy(x_hbm.at[pl.ds(base, b_per_w)], idx_v)
        start_gather(0, 0)
        start_gather(1, 1)
        wait_gather(0)
        scale(0)
        start_scatter(0, 0)

        def pair(i, carry):
            c = 2 * i + 1
            wait_scatter(0)
            start_gather(c + 1, 0)
            wait_gather(1)
            scale(1)
            start_scatter(c, 1)
            wait_scatter(1)
            start_gather(c + 2, 1)
            wait_gather(0)
            scale(0)
            start_scatter(c + 1, 0)
            return carry

        lax.fori_loop(0, (n_chunks - 2) // 2, pair, 0)

        wait_gather(1)
        scale(1)
        start_scatter(n_chunks - 1, 1)
        wait_scatter(0)
        wait_scatter(1)

    return emb_kernel(flat_idx, table)


def kernel(x, table):
    total = x.shape[0] * x.shape[1]
    flat = x.reshape(total).astype(jnp.int32)
    out = _emb(flat, table, total)
    return out.reshape(x.shape[0], x.shape[1], D_MODEL)
